# Initial kernel scaffold; baseline (speedup 1.0000x reference)
import jax
import jax.numpy as jnp
from jax.experimental import pallas as pl


def kernel(x, W_up, W_gate, W_down, W_pre, W_post, adapt_g, adapt_b, W_adapt_proj, Wa, ln_g, ln_b, W_expert_proj, W_output_proj, Wg, We):
    raise NotImplementedError("write your pallas kernel here")


# speedup vs baseline: 2.8686x; 2.8686x over previous
import functools

import jax
import jax.numpy as jnp
from jax.experimental import pallas as pl
from jax.experimental.pallas import tpu as pltpu

F32 = jnp.float32


def _silu(v):
    return v * jax.lax.logistic(v)


def _ln(v, g, b, eps=1e-5):
    mu = jnp.mean(v, axis=-1, keepdims=True)
    var = jnp.mean((v - mu) ** 2, axis=-1, keepdims=True)
    return (v - mu) / jnp.sqrt(var + eps) * g + b


def _dot(a, b):
    return jnp.dot(a, b, preferred_element_type=F32)


def _stage1_kernel(x_ref, wg_t_ref, wu_t_ref, wpost_t_ref, wpre_t_ref,
                   wr_ref, ag_ref, ab_ref,
                   hidden_ref, pre_ref, adapt_in_ref, adapt_out_ref,
                   logits_ref):
    xb = x_ref[...]
    gate = _dot(xb, wg_t_ref[...])
    up = _dot(xb, wu_t_ref[...])
    hid = _silu(gate) * up
    hidden_ref[...] = hid
    ag = ag_ref[...]
    ab = ab_ref[...]
    ao = _dot(hid, wpost_t_ref[...])
    adapt_out_ref[...] = _ln(ao, ag, ab)
    pr = _dot(xb, wpre_t_ref[...])
    pre_ref[...] = pr
    adapt_in_ref[...] = _ln(pr, ag, ab)
    logits_ref[...] = _dot(xb, wr_ref[...])


def _attn_kernel(q_ref, k_ref, v_ref, o_ref):
    q = q_ref[0]
    k = k_ref[0]
    v = v_ref[0]
    aw = jax.lax.dot_general(q, k, (((1,), (1,)), ((), ())),
                             preferred_element_type=F32)
    aw = _silu(jnp.clip(aw, -5.0, 5.0))
    o_ref[0] = _dot(aw, v)


def _m_kernel(a_ref, b_ref, m_ref):
    m_ref[...] = _dot(a_ref[...], b_ref[...])


def _router_weights(logits):
    lg = logits[:, 0:2]
    mg = jnp.max(lg, axis=1, keepdims=True)
    eg = jnp.exp(lg - mg)
    gp = eg / jnp.sum(eg, axis=1, keepdims=True)
    gp0 = gp[:, 0:1]
    gp1 = gp[:, 1:2]
    is_g1 = (gp1 > gp0).astype(F32)
    chosen_w = jnp.maximum(gp0, gp1)

    ll = logits[:, 2:6]
    ml = jnp.max(ll, axis=1, keepdims=True)
    el = jnp.exp(ll - ml)
    lp = el / jnp.sum(el, axis=1, keepdims=True)

    cols = [lp[:, j:j + 1] for j in range(4)]
    masks = []
    for j in range(4):
        rank = jnp.zeros_like(cols[j])
        for m in range(4):
            if m == j:
                continue
            gt = (cols[m] > cols[j]) if m > j else (cols[m] >= cols[j])
            rank = rank + gt.astype(F32)
        masks.append((rank < 2.0).astype(F32))
    sel = [cols[j] * masks[j] for j in range(4)]
    lsum = sel[0] + sel[1] + sel[2] + sel[3]
    inv = chosen_w / (lsum + 1e-7)
    fl = [s * inv for s in sel]

    g0 = 1.0 - is_g1
    ew = [fl[j] * g0 for j in range(4)] + [fl[j] * is_g1 for j in range(4)]
    tw = ew[0] + ew[1] + ew[2] + ew[3] + ew[4] + ew[5] + ew[6] + ew[7]
    return ew, tw


def _stage4_kernel(hidden_ref, adapt_ref, pre_ref, logits_ref,
                   wa_t_ref, lng_ref, lnb_ref,
                   wadapt_t_ref, wdown_t_ref, m_ref, o_ref):
    ew, tw = _router_weights(logits_ref[...])
    hid2 = hidden_ref[...] + 0.1 * _dot(adapt_ref[...], wadapt_t_ref[...])
    shared = _dot(hid2, wdown_t_ref[...])
    pre = pre_ref[...]
    g = None
    for i in range(8):
        h = _dot(pre, wa_t_ref[i])
        h = _ln(h, lng_ref[i:i + 1, :], lnb_ref[i:i + 1, :])
        term = ew[i] * h
        g = term if g is None else g + term
    o_ref[...] = shared * tw + 0.1 * _dot(g, m_ref[...])


def kernel(x, W_up, W_gate, W_down, W_pre, W_post, adapt_g, adapt_b,
           W_adapt_proj, Wa, ln_g, ln_b, W_expert_proj, W_output_proj,
           Wg, We):
    B, S, D = x.shape
    H = W_up.shape[0]
    A = W_pre.shape[0]
    E = Wa.shape[0]
    N = B * S

    xf = x.reshape(N, D)
    wg_t = W_gate.T
    wu_t = W_up.T
    wpost_t = W_post.T
    wpre_t = W_pre.T
    wr = jnp.concatenate([Wg, We, jnp.zeros((8 - Wg.shape[0] - We.shape[0], D),
                                            F32)], axis=0).T
    ag2 = adapt_g.reshape(1, A)
    ab2 = adapt_b.reshape(1, A)

    BM1 = 256
    grid1 = (N // BM1,)
    hidden, pre, adapt_in, adapt_out, logits = pl.pallas_call(
        _stage1_kernel,
        grid=grid1,
        in_specs=[
            pl.BlockSpec((BM1, D), lambda i: (i, 0)),
            pl.BlockSpec((D, H), lambda i: (0, 0)),
            pl.BlockSpec((D, H), lambda i: (0, 0)),
            pl.BlockSpec((H, A), lambda i: (0, 0)),
            pl.BlockSpec((D, A), lambda i: (0, 0)),
            pl.BlockSpec((D, 8), lambda i: (0, 0)),
            pl.BlockSpec((1, A), lambda i: (0, 0)),
            pl.BlockSpec((1, A), lambda i: (0, 0)),
        ],
        out_specs=[
            pl.BlockSpec((BM1, H), lambda i: (i, 0)),
            pl.BlockSpec((BM1, A), lambda i: (i, 0)),
            pl.BlockSpec((BM1, A), lambda i: (i, 0)),
            pl.BlockSpec((BM1, A), lambda i: (i, 0)),
            pl.BlockSpec((BM1, 8), lambda i: (i, 0)),
        ],
        out_shape=[
            jax.ShapeDtypeStruct((N, H), F32),
            jax.ShapeDtypeStruct((N, A), F32),
            jax.ShapeDtypeStruct((N, A), F32),
            jax.ShapeDtypeStruct((N, A), F32),
            jax.ShapeDtypeStruct((N, 8), F32),
        ],
        compiler_params=pltpu.CompilerParams(
            dimension_semantics=("arbitrary",)),
    )(xf, wg_t, wu_t, wpost_t, wpre_t, wr, ag2, ab2)

    ai3 = adapt_in.reshape(B, S, A)
    ao3 = adapt_out.reshape(B, S, A)
    BM2 = 512
    adapt = pl.pallas_call(
        _attn_kernel,
        grid=(B, S // BM2),
        in_specs=[
            pl.BlockSpec((1, BM2, A), lambda b, i: (b, i, 0)),
            pl.BlockSpec((1, S, A), lambda b, i: (b, 0, 0)),
            pl.BlockSpec((1, S, A), lambda b, i: (b, 0, 0)),
        ],
        out_specs=pl.BlockSpec((1, BM2, A), lambda b, i: (b, i, 0)),
        out_shape=jax.ShapeDtypeStruct((B, S, A), F32),
        compiler_params=pltpu.CompilerParams(
            dimension_semantics=("arbitrary", "arbitrary")),
    )(ai3, ao3, ai3)
    adapt = adapt.reshape(N, A)

    M = pl.pallas_call(
        _m_kernel,
        in_specs=[pl.BlockSpec((A, H), lambda: (0, 0)),
                  pl.BlockSpec((H, D), lambda: (0, 0))],
        out_specs=pl.BlockSpec((A, D), lambda: (0, 0)),
        out_shape=jax.ShapeDtypeStruct((A, D), F32),
    )(W_expert_proj.T, W_output_proj.T)

    wa_t = Wa.transpose(0, 2, 1)
    BM4 = 256
    out = pl.pallas_call(
        _stage4_kernel,
        grid=(N // BM4,),
        in_specs=[
            pl.BlockSpec((BM4, H), lambda i: (i, 0)),
            pl.BlockSpec((BM4, A), lambda i: (i, 0)),
            pl.BlockSpec((BM4, A), lambda i: (i, 0)),
            pl.BlockSpec((BM4, 8), lambda i: (i, 0)),
            pl.BlockSpec((E, A, A), lambda i: (0, 0, 0)),
            pl.BlockSpec((E, A), lambda i: (0, 0)),
            pl.BlockSpec((E, A), lambda i: (0, 0)),
            pl.BlockSpec((A, H), lambda i: (0, 0)),
            pl.BlockSpec((H, D), lambda i: (0, 0)),
            pl.BlockSpec((A, D), lambda i: (0, 0)),
        ],
        out_specs=pl.BlockSpec((BM4, D), lambda i: (i, 0)),
        out_shape=jax.ShapeDtypeStruct((N, D), F32),
        compiler_params=pltpu.CompilerParams(
            dimension_semantics=("arbitrary",)),
    )(hidden, adapt, pre, logits, wa_t, ln_g, ln_b,
      W_adapt_proj.T, W_down.T, M)

    return out.reshape(B, S, D)


# speedup vs baseline: 2.8721x; 1.0012x over previous
import functools

import jax
import jax.numpy as jnp
from jax.experimental import pallas as pl
from jax.experimental.pallas import tpu as pltpu

F32 = jnp.float32


def _silu(v):
    return v * jax.lax.logistic(v)


def _ln(v, g, b, eps=1e-5):
    mu = jnp.mean(v, axis=-1, keepdims=True)
    var = jnp.mean((v - mu) ** 2, axis=-1, keepdims=True)
    return (v - mu) / jnp.sqrt(var + eps) * g + b


def _dot(a, b):
    return jnp.dot(a, b, preferred_element_type=F32)


def _stage1_kernel(x_ref, wg_t_ref, wu_t_ref, wpost_t_ref, wpre_t_ref,
                   wr_ref, ag_ref, ab_ref,
                   hidden_ref, pre_ref, adapt_in_ref, adapt_out_ref,
                   logits_ref):
    xb = x_ref[...]
    xb16 = xb.astype(jnp.bfloat16)
    gate = _dot(xb16, wg_t_ref[...])
    up = _dot(xb16, wu_t_ref[...])
    hid = _silu(gate) * up
    hid16 = hid.astype(jnp.bfloat16)
    hidden_ref[...] = hid16
    ag = ag_ref[...]
    ab = ab_ref[...]
    ao = _dot(hid16, wpost_t_ref[...])
    adapt_out_ref[...] = _ln(ao, ag, ab)
    pr = _dot(xb16, wpre_t_ref[...])
    pre_ref[...] = pr
    adapt_in_ref[...] = _ln(pr, ag, ab)
    logits_ref[...] = _dot(xb, wr_ref[...])


def _attn_kernel(q_ref, k_ref, v_ref, o_ref):
    q = q_ref[0].astype(jnp.bfloat16)
    k = k_ref[0].astype(jnp.bfloat16)
    v = v_ref[0].astype(jnp.bfloat16)
    aw = jax.lax.dot_general(q, k, (((1,), (1,)), ((), ())),
                             preferred_element_type=F32)
    aw = _silu(jnp.clip(aw, -5.0, 5.0))
    o_ref[0] = _dot(aw.astype(jnp.bfloat16), v)


def _m_kernel(a_ref, b_ref, m_ref):
    m_ref[...] = _dot(a_ref[...], b_ref[...]).astype(jnp.bfloat16)


def _router_weights(logits):
    lg = logits[:, 0:2]
    mg = jnp.max(lg, axis=1, keepdims=True)
    eg = jnp.exp(lg - mg)
    gp = eg / jnp.sum(eg, axis=1, keepdims=True)
    gp0 = gp[:, 0:1]
    gp1 = gp[:, 1:2]
    is_g1 = (gp1 > gp0).astype(F32)
    chosen_w = jnp.maximum(gp0, gp1)

    ll = logits[:, 2:6]
    ml = jnp.max(ll, axis=1, keepdims=True)
    el = jnp.exp(ll - ml)
    lp = el / jnp.sum(el, axis=1, keepdims=True)

    cols = [lp[:, j:j + 1] for j in range(4)]
    masks = []
    for j in range(4):
        rank = jnp.zeros_like(cols[j])
        for m in range(4):
            if m == j:
                continue
            gt = (cols[m] > cols[j]) if m > j else (cols[m] >= cols[j])
            rank = rank + gt.astype(F32)
        masks.append((rank < 2.0).astype(F32))
    sel = [cols[j] * masks[j] for j in range(4)]
    lsum = sel[0] + sel[1] + sel[2] + sel[3]
    inv = chosen_w / (lsum + 1e-7)
    fl = [s * inv for s in sel]

    g0 = 1.0 - is_g1
    ew = [fl[j] * g0 for j in range(4)] + [fl[j] * is_g1 for j in range(4)]
    tw = ew[0] + ew[1] + ew[2] + ew[3] + ew[4] + ew[5] + ew[6] + ew[7]
    return ew, tw


def _stage4_kernel(hidden_ref, adapt_ref, pre_ref, logits_ref,
                   wa_t_ref, lng_ref, lnb_ref,
                   wadapt_t_ref, wdown_t_ref, m_ref, o_ref):
    ew, tw = _router_weights(logits_ref[...])
    hid2 = (hidden_ref[...].astype(F32)
            + 0.1 * _dot(adapt_ref[...].astype(jnp.bfloat16),
                         wadapt_t_ref[...]))
    shared = _dot(hid2.astype(jnp.bfloat16), wdown_t_ref[...])
    pre = pre_ref[...]
    g = None
    for i in range(8):
        h = _dot(pre, wa_t_ref[i])
        h = _ln(h, lng_ref[i:i + 1, :], lnb_ref[i:i + 1, :])
        term = ew[i] * h
        g = term if g is None else g + term
    o_ref[...] = shared * tw + 0.1 * _dot(g.astype(jnp.bfloat16),
                                          m_ref[...])


def kernel(x, W_up, W_gate, W_down, W_pre, W_post, adapt_g, adapt_b,
           W_adapt_proj, Wa, ln_g, ln_b, W_expert_proj, W_output_proj,
           Wg, We):
    B, S, D = x.shape
    H = W_up.shape[0]
    A = W_pre.shape[0]
    E = Wa.shape[0]
    N = B * S

    xf = x.reshape(N, D)
    wg_t = W_gate.T.astype(jnp.bfloat16)
    wu_t = W_up.T.astype(jnp.bfloat16)
    wpost_t = W_post.T.astype(jnp.bfloat16)
    wpre_t = W_pre.T.astype(jnp.bfloat16)
    wr = jnp.concatenate([Wg, We, jnp.zeros((8 - Wg.shape[0] - We.shape[0], D),
                                            F32)], axis=0).T
    ag2 = adapt_g.reshape(1, A)
    ab2 = adapt_b.reshape(1, A)

    BM1 = 256
    grid1 = (N // BM1,)
    hidden, pre, adapt_in, adapt_out, logits = pl.pallas_call(
        _stage1_kernel,
        grid=grid1,
        in_specs=[
            pl.BlockSpec((BM1, D), lambda i: (i, 0)),
            pl.BlockSpec((D, H), lambda i: (0, 0)),
            pl.BlockSpec((D, H), lambda i: (0, 0)),
            pl.BlockSpec((H, A), lambda i: (0, 0)),
            pl.BlockSpec((D, A), lambda i: (0, 0)),
            pl.BlockSpec((D, 8), lambda i: (0, 0)),
            pl.BlockSpec((1, A), lambda i: (0, 0)),
            pl.BlockSpec((1, A), lambda i: (0, 0)),
        ],
        out_specs=[
            pl.BlockSpec((BM1, H), lambda i: (i, 0)),
            pl.BlockSpec((BM1, A), lambda i: (i, 0)),
            pl.BlockSpec((BM1, A), lambda i: (i, 0)),
            pl.BlockSpec((BM1, A), lambda i: (i, 0)),
            pl.BlockSpec((BM1, 8), lambda i: (i, 0)),
        ],
        out_shape=[
            jax.ShapeDtypeStruct((N, H), jnp.bfloat16),
            jax.ShapeDtypeStruct((N, A), F32),
            jax.ShapeDtypeStruct((N, A), F32),
            jax.ShapeDtypeStruct((N, A), F32),
            jax.ShapeDtypeStruct((N, 8), F32),
        ],
        compiler_params=pltpu.CompilerParams(
            dimension_semantics=("arbitrary",)),
    )(xf, wg_t, wu_t, wpost_t, wpre_t, wr, ag2, ab2)

    ai3 = adapt_in.reshape(B, S, A)
    ao3 = adapt_out.reshape(B, S, A)
    BM2 = 512
    adapt = pl.pallas_call(
        _attn_kernel,
        grid=(B, S // BM2),
        in_specs=[
            pl.BlockSpec((1, BM2, A), lambda b, i: (b, i, 0)),
            pl.BlockSpec((1, S, A), lambda b, i: (b, 0, 0)),
            pl.BlockSpec((1, S, A), lambda b, i: (b, 0, 0)),
        ],
        out_specs=pl.BlockSpec((1, BM2, A), lambda b, i: (b, i, 0)),
        out_shape=jax.ShapeDtypeStruct((B, S, A), F32),
        compiler_params=pltpu.CompilerParams(
            dimension_semantics=("arbitrary", "arbitrary")),
    )(ai3, ao3, ai3)
    adapt = adapt.reshape(N, A)

    M = pl.pallas_call(
        _m_kernel,
        in_specs=[pl.BlockSpec((A, H), lambda: (0, 0)),
                  pl.BlockSpec((H, D), lambda: (0, 0))],
        out_specs=pl.BlockSpec((A, D), lambda: (0, 0)),
        out_shape=jax.ShapeDtypeStruct((A, D), jnp.bfloat16),
    )(W_expert_proj.T, W_output_proj.T)

    wa_t = Wa.transpose(0, 2, 1)
    BM4 = 256
    out = pl.pallas_call(
        _stage4_kernel,
        grid=(N // BM4,),
        in_specs=[
            pl.BlockSpec((BM4, H), lambda i: (i, 0)),
            pl.BlockSpec((BM4, A), lambda i: (i, 0)),
            pl.BlockSpec((BM4, A), lambda i: (i, 0)),
            pl.BlockSpec((BM4, 8), lambda i: (i, 0)),
            pl.BlockSpec((E, A, A), lambda i: (0, 0, 0)),
            pl.BlockSpec((E, A), lambda i: (0, 0)),
            pl.BlockSpec((E, A), lambda i: (0, 0)),
            pl.BlockSpec((A, H), lambda i: (0, 0)),
            pl.BlockSpec((H, D), lambda i: (0, 0)),
            pl.BlockSpec((A, D), lambda i: (0, 0)),
        ],
        out_specs=pl.BlockSpec((BM4, D), lambda i: (i, 0)),
        out_shape=jax.ShapeDtypeStruct((N, D), F32),
        compiler_params=pltpu.CompilerParams(
            dimension_semantics=("arbitrary",)),
    )(hidden, adapt, pre, logits, wa_t, ln_g, ln_b,
      W_adapt_proj.T.astype(jnp.bfloat16), W_down.T.astype(jnp.bfloat16), M)

    return out.reshape(B, S, D)


# speedup vs baseline: 3.4539x; 1.2026x over previous
import jax
import jax.numpy as jnp
from jax.experimental import pallas as pl
from jax.experimental.pallas import tpu as pltpu

F32 = jnp.float32
BF16 = jnp.bfloat16

_NT = (((1,), (1,)), ((), ()))
_NN = (((1,), (0,)), ((), ()))


def _silu(v):
    return v * jax.lax.logistic(v)


def _ln(v, g, b, eps=1e-5):
    mu = jnp.mean(v, axis=-1, keepdims=True)
    var = jnp.mean((v - mu) ** 2, axis=-1, keepdims=True)
    return (v - mu) / jnp.sqrt(var + eps) * g + b


def _dg(a, b, dn):
    return jax.lax.dot_general(a, b, dn, preferred_element_type=F32)


def _stage1_kernel(x_ref, wg_ref, wu_ref, wpost_ref, wpre_ref,
                   wr_ref, ag_ref, ab_ref,
                   hidden_ref, pre_ref, adapt_in_ref, adapt_out_ref,
                   logits_ref):
    xb = x_ref[...]
    xb16 = xb.astype(BF16)
    gate = _dg(xb16, wg_ref[...], _NT)
    up = _dg(xb16, wu_ref[...], _NT)
    hid = _silu(gate) * up
    hid16 = hid.astype(BF16)
    hidden_ref[...] = hid16
    ag = ag_ref[...]
    ab = ab_ref[...]
    ao = _dg(hid16, wpost_ref[...], _NT)
    adapt_out_ref[...] = _ln(ao, ag, ab)
    pr = _dg(xb16, wpre_ref[...], _NT)
    pre_ref[...] = pr
    adapt_in_ref[...] = _ln(pr, ag, ab)
    logits_ref[...] = _dg(xb, wr_ref[...], _NT)


def _attn_kernel(q_ref, k_ref, v_ref, o_ref):
    q = q_ref[0].astype(BF16)
    k = k_ref[0].astype(BF16)
    v = v_ref[0].astype(BF16)
    aw = _dg(q, k, _NT)
    aw = _silu(jnp.clip(aw, -5.0, 5.0))
    o_ref[0] = _dg(aw.astype(BF16), v, _NN)


def _m_kernel(a_ref, b_ref, m_ref):
    m = jax.lax.dot_general(a_ref[...], b_ref[...], (((0,), (1,)), ((), ())),
                            preferred_element_type=F32)
    m_ref[...] = m.astype(BF16)


def _router_weights(logits):
    lg = logits[:, 0:2]
    mg = jnp.max(lg, axis=1, keepdims=True)
    eg = jnp.exp(lg - mg)
    gp = eg / jnp.sum(eg, axis=1, keepdims=True)
    gp0 = gp[:, 0:1]
    gp1 = gp[:, 1:2]
    is_g1 = (gp1 > gp0).astype(F32)
    chosen_w = jnp.maximum(gp0, gp1)

    ll = logits[:, 2:6]
    ml = jnp.max(ll, axis=1, keepdims=True)
    el = jnp.exp(ll - ml)
    lp = el / jnp.sum(el, axis=1, keepdims=True)

    cols = [lp[:, j:j + 1] for j in range(4)]
    masks = []
    for j in range(4):
        rank = jnp.zeros_like(cols[j])
        for m in range(4):
            if m == j:
                continue
            gt = (cols[m] > cols[j]) if m > j else (cols[m] >= cols[j])
            rank = rank + gt.astype(F32)
        masks.append((rank < 2.0).astype(F32))
    sel = [cols[j] * masks[j] for j in range(4)]
    lsum = sel[0] + sel[1] + sel[2] + sel[3]
    inv = chosen_w / (lsum + 1e-7)
    fl = [s * inv for s in sel]

    g0 = 1.0 - is_g1
    ew = [fl[j] * g0 for j in range(4)] + [fl[j] * is_g1 for j in range(4)]
    tw = ew[0] + ew[1] + ew[2] + ew[3] + ew[4] + ew[5] + ew[6] + ew[7]
    return ew, tw


def _stage4_kernel(hidden_ref, adapt_ref, pre_ref, logits_ref,
                   wa_ref, lng_ref, lnb_ref,
                   wadapt_ref, wdown_ref, m_ref, o_ref):
    ew, tw = _router_weights(logits_ref[...])
    hid2 = (hidden_ref[...].astype(F32)
            + 0.1 * _dg(adapt_ref[...].astype(BF16), wadapt_ref[...], _NT))
    shared = _dg(hid2.astype(BF16), wdown_ref[...], _NT)
    pre = pre_ref[...]
    g = None
    for i in range(8):
        h = _dg(pre, wa_ref[i], _NT)
        h = _ln(h, lng_ref[i:i + 1, :], lnb_ref[i:i + 1, :])
        term = ew[i] * h
        g = term if g is None else g + term
    o_ref[...] = shared * tw + 0.1 * _dg(g.astype(BF16), m_ref[...], _NN)


def kernel(x, W_up, W_gate, W_down, W_pre, W_post, adapt_g, adapt_b,
           W_adapt_proj, Wa, ln_g, ln_b, W_expert_proj, W_output_proj,
           Wg, We):
    B, S, D = x.shape
    H = W_up.shape[0]
    A = W_pre.shape[0]
    E = Wa.shape[0]
    N = B * S

    xf = x.reshape(N, D)
    wg16 = W_gate.astype(BF16)
    wu16 = W_up.astype(BF16)
    wpost16 = W_post.astype(BF16)
    wpre16 = W_pre.astype(BF16)
    wr = jnp.concatenate(
        [Wg, We, jnp.zeros((8 - Wg.shape[0] - We.shape[0], D), F32)],
        axis=0)
    ag2 = adapt_g.reshape(1, A)
    ab2 = adapt_b.reshape(1, A)

    BM1 = 512
    grid1 = (N // BM1,)
    hidden, pre, adapt_in, adapt_out, logits = pl.pallas_call(
        _stage1_kernel,
        grid=grid1,
        in_specs=[
            pl.BlockSpec((BM1, D), lambda i: (i, 0)),
            pl.BlockSpec((H, D), lambda i: (0, 0)),
            pl.BlockSpec((H, D), lambda i: (0, 0)),
            pl.BlockSpec((A, H), lambda i: (0, 0)),
            pl.BlockSpec((A, D), lambda i: (0, 0)),
            pl.BlockSpec((8, D), lambda i: (0, 0)),
            pl.BlockSpec((1, A), lambda i: (0, 0)),
            pl.BlockSpec((1, A), lambda i: (0, 0)),
        ],
        out_specs=[
            pl.BlockSpec((BM1, H), lambda i: (i, 0)),
            pl.BlockSpec((BM1, A), lambda i: (i, 0)),
            pl.BlockSpec((BM1, A), lambda i: (i, 0)),
            pl.BlockSpec((BM1, A), lambda i: (i, 0)),
            pl.BlockSpec((BM1, 8), lambda i: (i, 0)),
        ],
        out_shape=[
            jax.ShapeDtypeStruct((N, H), BF16),
            jax.ShapeDtypeStruct((N, A), F32),
            jax.ShapeDtypeStruct((N, A), F32),
            jax.ShapeDtypeStruct((N, A), F32),
            jax.ShapeDtypeStruct((N, 8), F32),
        ],
        compiler_params=pltpu.CompilerParams(
            dimension_semantics=("arbitrary",)),
    )(xf, wg16, wu16, wpost16, wpre16, wr, ag2, ab2)

    ai3 = adapt_in.reshape(B, S, A)
    ao3 = adapt_out.reshape(B, S, A)
    BM2 = 512
    adapt = pl.pallas_call(
        _attn_kernel,
        grid=(B, S // BM2),
        in_specs=[
            pl.BlockSpec((1, BM2, A), lambda b, i: (b, i, 0)),
            pl.BlockSpec((1, S, A), lambda b, i: (b, 0, 0)),
            pl.BlockSpec((1, S, A), lambda b, i: (b, 0, 0)),
        ],
        out_specs=pl.BlockSpec((1, BM2, A), lambda b, i: (b, i, 0)),
        out_shape=jax.ShapeDtypeStruct((B, S, A), F32),
        compiler_params=pltpu.CompilerParams(
            dimension_semantics=("arbitrary", "arbitrary")),
    )(ai3, ao3, ai3)
    adapt = adapt.reshape(N, A)

    M = pl.pallas_call(
        _m_kernel,
        in_specs=[pl.BlockSpec((H, A), lambda: (0, 0)),
                  pl.BlockSpec((D, H), lambda: (0, 0))],
        out_specs=pl.BlockSpec((A, D), lambda: (0, 0)),
        out_shape=jax.ShapeDtypeStruct((A, D), BF16),
    )(W_expert_proj, W_output_proj)

    BM4 = 512
    out = pl.pallas_call(
        _stage4_kernel,
        grid=(N // BM4,),
        in_specs=[
            pl.BlockSpec((BM4, H), lambda i: (i, 0)),
            pl.BlockSpec((BM4, A), lambda i: (i, 0)),
            pl.BlockSpec((BM4, A), lambda i: (i, 0)),
            pl.BlockSpec((BM4, 8), lambda i: (i, 0)),
            pl.BlockSpec((E, A, A), lambda i: (0, 0, 0)),
            pl.BlockSpec((E, A), lambda i: (0, 0)),
            pl.BlockSpec((E, A), lambda i: (0, 0)),
            pl.BlockSpec((H, A), lambda i: (0, 0)),
            pl.BlockSpec((D, H), lambda i: (0, 0)),
            pl.BlockSpec((A, D), lambda i: (0, 0)),
        ],
        out_specs=pl.BlockSpec((BM4, D), lambda i: (i, 0)),
        out_shape=jax.ShapeDtypeStruct((N, D), F32),
        compiler_params=pltpu.CompilerParams(
            dimension_semantics=("arbitrary",)),
    )(hidden, adapt, pre, logits, Wa, ln_g, ln_b,
      W_adapt_proj.astype(BF16), W_down.astype(BF16), M)

    return out.reshape(B, S, D)


# speedup vs baseline: 3.4561x; 1.0006x over previous
import jax
import jax.numpy as jnp
from jax.experimental import pallas as pl
from jax.experimental.pallas import tpu as pltpu

F32 = jnp.float32
BF16 = jnp.bfloat16

_NT = (((1,), (1,)), ((), ()))
_NN = (((1,), (0,)), ((), ()))


def _silu(v):
    return v * jax.lax.logistic(v)


def _ln(v, g, b, eps=1e-5):
    mu = jnp.mean(v, axis=-1, keepdims=True)
    var = jnp.mean((v - mu) ** 2, axis=-1, keepdims=True)
    return (v - mu) / jnp.sqrt(var + eps) * g + b


def _dg(a, b, dn):
    return jax.lax.dot_general(a, b, dn, preferred_element_type=F32)


def _stage1_kernel(x_ref, wg_ref, wu_ref, wpost_ref, wpre_ref,
                   wr_ref, ag_ref, ab_ref,
                   hidden_ref, pre_ref, adapt_in_ref, adapt_out_ref,
                   logits_ref):
    xb = x_ref[...]
    xb16 = xb.astype(BF16)
    gate = _dg(xb16, wg_ref[...], _NT)
    up = _dg(xb16, wu_ref[...], _NT)
    hid = _silu(gate) * up
    hid16 = hid.astype(BF16)
    hidden_ref[...] = hid16
    ag = ag_ref[...]
    ab = ab_ref[...]
    ao = _dg(hid16, wpost_ref[...], _NT)
    adapt_out_ref[...] = _ln(ao, ag, ab)
    pr = _dg(xb16, wpre_ref[...], _NT)
    pre_ref[...] = pr
    adapt_in_ref[...] = _ln(pr, ag, ab)
    logits_ref[...] = _dg(xb, wr_ref[...], _NT)


def _attn_kernel(q_ref, k_ref, v_ref, o_ref):
    q = q_ref[0].astype(BF16)
    k = k_ref[0].astype(BF16)
    v = v_ref[0].astype(BF16)
    aw = _dg(q, k, _NT)
    aw = _silu(jnp.clip(aw, -5.0, 5.0))
    o_ref[0] = _dg(aw.astype(BF16), v, _NN)


def _m_kernel(a_ref, b_ref, m_ref):
    m = jax.lax.dot_general(a_ref[...], b_ref[...], (((0,), (1,)), ((), ())),
                            preferred_element_type=F32)
    m_ref[...] = m.astype(BF16)


def _router_weights(logits):
    lg = logits[:, 0:2]
    mg = jnp.max(lg, axis=1, keepdims=True)
    eg = jnp.exp(lg - mg)
    gp = eg / jnp.sum(eg, axis=1, keepdims=True)
    gp0 = gp[:, 0:1]
    gp1 = gp[:, 1:2]
    is_g1 = (gp1 > gp0).astype(F32)
    chosen_w = jnp.maximum(gp0, gp1)

    ll = logits[:, 2:6]
    ml = jnp.max(ll, axis=1, keepdims=True)
    el = jnp.exp(ll - ml)
    lp = el / jnp.sum(el, axis=1, keepdims=True)

    cols = [lp[:, j:j + 1] for j in range(4)]
    masks = []
    for j in range(4):
        rank = jnp.zeros_like(cols[j])
        for m in range(4):
            if m == j:
                continue
            gt = (cols[m] > cols[j]) if m > j else (cols[m] >= cols[j])
            rank = rank + gt.astype(F32)
        masks.append((rank < 2.0).astype(F32))
    sel = [cols[j] * masks[j] for j in range(4)]
    lsum = sel[0] + sel[1] + sel[2] + sel[3]
    inv = chosen_w / (lsum + 1e-7)
    fl = [s * inv for s in sel]

    g0 = 1.0 - is_g1
    ew = [fl[j] * g0 for j in range(4)] + [fl[j] * is_g1 for j in range(4)]
    tw = ew[0] + ew[1] + ew[2] + ew[3] + ew[4] + ew[5] + ew[6] + ew[7]
    return ew, tw


def _stage4_kernel(hidden_ref, adapt_ref, pre_ref, logits_ref,
                   wa_ref, lng_ref, lnb_ref,
                   wadapt_ref, wdown_ref, m_ref, o_ref):
    ew, tw = _router_weights(logits_ref[...])
    hid2 = (hidden_ref[...].astype(F32)
            + 0.1 * _dg(adapt_ref[...].astype(BF16), wadapt_ref[...], _NT))
    shared = _dg(hid2.astype(BF16), wdown_ref[...], _NT)
    pre = pre_ref[...]
    g = None
    for i in range(8):
        h = _dg(pre, wa_ref[i], _NT)
        h = _ln(h, lng_ref[i:i + 1, :], lnb_ref[i:i + 1, :])
        term = ew[i] * h
        g = term if g is None else g + term
    o_ref[...] = shared * tw + 0.1 * _dg(g.astype(BF16), m_ref[...], _NN)


def kernel(x, W_up, W_gate, W_down, W_pre, W_post, adapt_g, adapt_b,
           W_adapt_proj, Wa, ln_g, ln_b, W_expert_proj, W_output_proj,
           Wg, We):
    B, S, D = x.shape
    H = W_up.shape[0]
    A = W_pre.shape[0]
    E = Wa.shape[0]
    N = B * S

    xf = x.reshape(N, D)
    wg16 = W_gate.astype(BF16)
    wu16 = W_up.astype(BF16)
    wpost16 = W_post.astype(BF16)
    wpre16 = W_pre.astype(BF16)
    wr = jnp.concatenate(
        [Wg, We, jnp.zeros((8 - Wg.shape[0] - We.shape[0], D), F32)],
        axis=0)
    ag2 = adapt_g.reshape(1, A)
    ab2 = adapt_b.reshape(1, A)

    BM1 = 512
    grid1 = (N // BM1,)
    hidden, pre, adapt_in, adapt_out, logits = pl.pallas_call(
        _stage1_kernel,
        grid=grid1,
        in_specs=[
            pl.BlockSpec((BM1, D), lambda i: (i, 0)),
            pl.BlockSpec((H, D), lambda i: (0, 0)),
            pl.BlockSpec((H, D), lambda i: (0, 0)),
            pl.BlockSpec((A, H), lambda i: (0, 0)),
            pl.BlockSpec((A, D), lambda i: (0, 0)),
            pl.BlockSpec((8, D), lambda i: (0, 0)),
            pl.BlockSpec((1, A), lambda i: (0, 0)),
            pl.BlockSpec((1, A), lambda i: (0, 0)),
        ],
        out_specs=[
            pl.BlockSpec((BM1, H), lambda i: (i, 0)),
            pl.BlockSpec((BM1, A), lambda i: (i, 0)),
            pl.BlockSpec((BM1, A), lambda i: (i, 0)),
            pl.BlockSpec((BM1, A), lambda i: (i, 0)),
            pl.BlockSpec((BM1, 8), lambda i: (i, 0)),
        ],
        out_shape=[
            jax.ShapeDtypeStruct((N, H), BF16),
            jax.ShapeDtypeStruct((N, A), F32),
            jax.ShapeDtypeStruct((N, A), F32),
            jax.ShapeDtypeStruct((N, A), F32),
            jax.ShapeDtypeStruct((N, 8), F32),
        ],
        compiler_params=pltpu.CompilerParams(
            dimension_semantics=("parallel",)),
    )(xf, wg16, wu16, wpost16, wpre16, wr, ag2, ab2)

    ai3 = adapt_in.reshape(B, S, A)
    ao3 = adapt_out.reshape(B, S, A)
    BM2 = 512
    adapt = pl.pallas_call(
        _attn_kernel,
        grid=(B, S // BM2),
        in_specs=[
            pl.BlockSpec((1, BM2, A), lambda b, i: (b, i, 0)),
            pl.BlockSpec((1, S, A), lambda b, i: (b, 0, 0)),
            pl.BlockSpec((1, S, A), lambda b, i: (b, 0, 0)),
        ],
        out_specs=pl.BlockSpec((1, BM2, A), lambda b, i: (b, i, 0)),
        out_shape=jax.ShapeDtypeStruct((B, S, A), F32),
        compiler_params=pltpu.CompilerParams(
            dimension_semantics=("parallel", "parallel")),
    )(ai3, ao3, ai3)
    adapt = adapt.reshape(N, A)

    M = pl.pallas_call(
        _m_kernel,
        in_specs=[pl.BlockSpec((H, A), lambda: (0, 0)),
                  pl.BlockSpec((D, H), lambda: (0, 0))],
        out_specs=pl.BlockSpec((A, D), lambda: (0, 0)),
        out_shape=jax.ShapeDtypeStruct((A, D), BF16),
    )(W_expert_proj, W_output_proj)

    BM4 = 512
    out = pl.pallas_call(
        _stage4_kernel,
        grid=(N // BM4,),
        in_specs=[
            pl.BlockSpec((BM4, H), lambda i: (i, 0)),
            pl.BlockSpec((BM4, A), lambda i: (i, 0)),
            pl.BlockSpec((BM4, A), lambda i: (i, 0)),
            pl.BlockSpec((BM4, 8), lambda i: (i, 0)),
            pl.BlockSpec((E, A, A), lambda i: (0, 0, 0)),
            pl.BlockSpec((E, A), lambda i: (0, 0)),
            pl.BlockSpec((E, A), lambda i: (0, 0)),
            pl.BlockSpec((H, A), lambda i: (0, 0)),
            pl.BlockSpec((D, H), lambda i: (0, 0)),
            pl.BlockSpec((A, D), lambda i: (0, 0)),
        ],
        out_specs=pl.BlockSpec((BM4, D), lambda i: (i, 0)),
        out_shape=jax.ShapeDtypeStruct((N, D), F32),
        compiler_params=pltpu.CompilerParams(
            dimension_semantics=("parallel",)),
    )(hidden, adapt, pre, logits, Wa, ln_g, ln_b,
      W_adapt_proj.astype(BF16), W_down.astype(BF16), M)

    return out.reshape(B, S, D)
